# Initial kernel scaffold; baseline (speedup 1.0000x reference)
import jax
import jax.numpy as jnp
from jax.experimental import pallas as pl


def kernel(x, embed_weight, pe):
    raise NotImplementedError("write your pallas kernel here")


# speedup vs baseline: 3.3051x; 3.3051x over previous
import functools

import jax
import jax.numpy as jnp
from jax import lax
from jax.experimental import pallas as pl
from jax.experimental.pallas import tpu as pltpu
from jax.experimental.pallas import tpu_sc as plsc

VOCAB = 100000
D = 64
L = 200
B = 4096

NC = 2
NS = 16
NW = NC * NS
BPW = B // NW
C0 = 104
C1 = L - C0


def _body(x_hbm, table_hbm, pe_hbm, out_hbm, idx_v, pe_v, buf_v, gsem):
    wid = lax.axis_index("s") * NC + lax.axis_index("c")
    row0 = wid * BPW

    pltpu.sync_copy(pe_hbm, pe_v)
    pltpu.sync_copy(x_hbm.at[pl.ds(row0 * L, BPW * L)], idx_v)

    def batch_body(g, carry):
        c0 = pltpu.async_copy(
            table_hbm.at[idx_v.at[pl.ds(g * L, C0)]],
            buf_v.at[pl.ds(0, C0)], gsem)
        c1 = pltpu.async_copy(
            table_hbm.at[idx_v.at[pl.ds(g * L + C0, C1)]],
            buf_v.at[pl.ds(C0, C1)], gsem)
        c0.wait()
        c1.wait()

        def add_body(r4, c2):
            for u in range(4):
                r = r4 * 4 + u
                for c in range(D // 16):
                    sl = pl.ds(c * 16, 16)
                    buf_v[r, sl] = buf_v[r, sl] + pe_v[r, sl]
            return c2

        lax.fori_loop(0, L // 4, add_body, 0)

        pltpu.sync_copy(buf_v, out_hbm.at[pl.ds((row0 + g) * L, L)])
        return carry

    lax.fori_loop(0, BPW, batch_body, 0)


@jax.jit
def _run(x, table, pe):
    mesh = plsc.VectorSubcoreMesh(core_axis_name="c", subcore_axis_name="s")
    kfn = functools.partial(
        pl.kernel,
        mesh=mesh,
        compiler_params=pltpu.CompilerParams(use_tc_tiling_on_sc=False),
        out_type=jax.ShapeDtypeStruct((B * L, D), jnp.float32),
        scratch_types=[
            pltpu.VMEM((BPW * L,), jnp.int32),
            pltpu.VMEM((L, D), jnp.float32),
            pltpu.VMEM((L, D), jnp.float32),
            pltpu.SemaphoreType.DMA,
        ],
    )(_body)
    return kfn(x.reshape(B * L), table, pe)


def kernel(x, embed_weight, pe):
    out = _run(x, embed_weight, pe)
    return out.reshape(B, L, D)


# speedup vs baseline: 4.2554x; 1.2875x over previous
import functools

import jax
import jax.numpy as jnp
from jax import lax
from jax.experimental import pallas as pl
from jax.experimental.pallas import tpu as pltpu
from jax.experimental.pallas import tpu_sc as plsc

VOCAB = 100000
D = 64
L = 200
B = 4096

NC = 2
NS = 16
NW = NC * NS
BPW = B // NW
C0 = 104
C1 = L - C0
NSLOT = 4


def _body(x_hbm, table_hbm, pe_hbm, out_hbm, idx_v, pe_v, buf_v, gsem, ssem):
    wid = lax.axis_index("s") * NC + lax.axis_index("c")
    row0 = wid * BPW

    pltpu.sync_copy(pe_hbm, pe_v)
    pltpu.sync_copy(x_hbm.at[pl.ds(row0 * L, BPW * L)], idx_v)

    def start_gather(g, k):
        pltpu.async_copy(
            table_hbm.at[idx_v.at[pl.ds(g * L, C0)]],
            buf_v.at[k, pl.ds(0, C0)], gsem)
        pltpu.async_copy(
            table_hbm.at[idx_v.at[pl.ds(g * L + C0, C1)]],
            buf_v.at[k, pl.ds(C0, C1)], gsem)

    def wait_gather(k):
        pltpu.make_async_copy(
            out_hbm.at[pl.ds(0, L)], buf_v.at[k], gsem).wait()

    def start_scatter(g, k):
        pltpu.async_copy(
            buf_v.at[k], out_hbm.at[pl.ds((row0 + g) * L, L)], ssem)

    def wait_scatter(k):
        pltpu.make_async_copy(
            buf_v.at[k], out_hbm.at[pl.ds(0, L)], ssem).wait()

    def add_pe(k):
        def add_body(r4, carry):
            for u in range(4):
                r = r4 * 4 + u
                for c in range(D // 16):
                    sl = pl.ds(c * 16, 16)
                    buf_v[k, r, sl] = buf_v[k, r, sl] + pe_v[r, sl]
            return carry

        lax.fori_loop(0, L // 4, add_body, 0)

    start_gather(0, 0)
    start_gather(1, 1)

    def outer_body(g4, carry):
        for k in range(NSLOT):
            g = g4 * NSLOT + k
            if k < 2:
                @pl.when(g4 > 0)
                def _():
                    wait_scatter((k + 2) % NSLOT)
                @pl.when(g4 > 0)
                def _():
                    start_gather(g + 2, (k + 2) % NSLOT)
                @pl.when(g4 == 0)
                def _():
                    start_gather(g + 2, (k + 2) % NSLOT)
            else:
                wait_scatter((k + 2) % NSLOT)

                @pl.when(g4 < (BPW // NSLOT) - 1)
                def _():
                    start_gather(g + 2, (k + 2) % NSLOT)
            wait_gather(k)
            add_pe(k)
            start_scatter(g, k)
        return carry

    lax.fori_loop(0, BPW // NSLOT, outer_body, 0)

    wait_scatter((BPW - 2) % NSLOT)
    wait_scatter((BPW - 1) % NSLOT)


@jax.jit
def _run(x, table, pe):
    mesh = plsc.VectorSubcoreMesh(core_axis_name="c", subcore_axis_name="s")
    kfn = functools.partial(
        pl.kernel,
        mesh=mesh,
        compiler_params=pltpu.CompilerParams(use_tc_tiling_on_sc=False),
        out_type=jax.ShapeDtypeStruct((B * L, D), jnp.float32),
        scratch_types=[
            pltpu.VMEM((BPW * L,), jnp.int32),
            pltpu.VMEM((L, D), jnp.float32),
            pltpu.VMEM((NSLOT, L, D), jnp.float32),
            pltpu.SemaphoreType.DMA,
            pltpu.SemaphoreType.DMA,
        ],
    )(_body)
    return kfn(x.reshape(B * L), table, pe)


def kernel(x, embed_weight, pe):
    out = _run(x, embed_weight, pe)
    return out.reshape(B, L, D)
